# baseline (speedup 1.0000x reference)
import functools

import jax
import jax.numpy as jnp
from jax import lax
from jax.experimental import pallas as pl
from jax.experimental.pallas import tpu as pltpu
from jax.experimental.pallas import tpu_sc as plsc

BATCH = 16384
FIELDS = 26
DIM = 64
NROWS = BATCH * FIELDS

_INFO = plsc.get_sparse_core_info()
NC = _INFO.num_cores
NS = _INFO.num_subcores
NW = NC * NS
ROWS_PER_W = NROWS // NW
CHUNK = 1024
NCHUNK = ROWS_PER_W // CHUNK


def _gather_sc(idx_flat, table):
    mesh = plsc.VectorSubcoreMesh(core_axis_name="c", subcore_axis_name="s")

    @functools.partial(
        pl.kernel,
        mesh=mesh,
        compiler_params=pltpu.CompilerParams(use_tc_tiling_on_sc=False),
        out_type=jax.ShapeDtypeStruct((NROWS, DIM), jnp.float32),
        scratch_types=[
            pltpu.VMEM((ROWS_PER_W,), jnp.int32),
            pltpu.VMEM((CHUNK, DIM), jnp.float32),
            pltpu.SemaphoreType.DMA,
        ],
    )
    def k(idx_hbm, table_hbm, out_hbm, idx_v, rows_v, sem):
        wid = lax.axis_index("s") * NC + lax.axis_index("c")
        base = wid * ROWS_PER_W
        pltpu.sync_copy(idx_hbm.at[pl.ds(base, ROWS_PER_W)], idx_v)
        for j in range(NCHUNK):
            off = j * CHUNK
            pltpu.async_copy(
                table_hbm.at[idx_v.at[pl.ds(off, CHUNK)]], rows_v, sem
            ).wait()
            pltpu.sync_copy(rows_v, out_hbm.at[pl.ds(base + off, CHUNK)])

    return k(idx_flat, table)


def _mm_body(x_ref, w_ref, b_ref, o_ref):
    o_ref[...] = (
        jnp.dot(x_ref[...], w_ref[...], preferred_element_type=jnp.float32)
        + b_ref[...]
    )


def _project_tc(x, W, b):
    blk = 2048
    return pl.pallas_call(
        _mm_body,
        grid=(NROWS // blk,),
        in_specs=[
            pl.BlockSpec((blk, DIM), lambda i: (i, 0)),
            pl.BlockSpec((DIM, DIM), lambda i: (0, 0)),
            pl.BlockSpec((1, DIM), lambda i: (0, 0)),
        ],
        out_specs=pl.BlockSpec((blk, DIM), lambda i: (i, 0)),
        out_shape=jax.ShapeDtypeStruct((NROWS, DIM), jnp.float32),
    )(x, W, b.reshape(1, DIM))


def kernel(indices, table, W, b):
    idx_flat = indices.reshape(-1).astype(jnp.int32)
    feats = _gather_sc(idx_flat, table)
    out = _project_tc(feats, W, b)
    return out.reshape(BATCH, FIELDS, DIM)


# speedup vs baseline: 2.3602x; 2.3602x over previous
import functools

import jax
import jax.numpy as jnp
from jax import lax
from jax.experimental import pallas as pl
from jax.experimental.pallas import tpu as pltpu
from jax.experimental.pallas import tpu_sc as plsc

BATCH = 16384
FIELDS = 26
DIM = 64
VOCAB = 1000000
NROWS = BATCH * FIELDS
NPAIR = NROWS // 2

_INFO = plsc.get_sparse_core_info()
NC = _INFO.num_cores
NS = _INFO.num_subcores
NW = NC * NS
ROWS_PER_W = NROWS // NW
CHUNK = 512
NCHUNK = ROWS_PER_W // CHUNK


def _project_body(xt_ref, w_ref, b_ref, o_ref):
    y = jax.lax.dot_general(
        xt_ref[...], w_ref[...], (((0,), (0,)), ((), ())),
        preferred_element_type=jnp.float32,
    )
    o_ref[:, 0:DIM] = y + b_ref[...]
    o_ref[:, DIM:2 * DIM] = jnp.zeros_like(y)


def _project_tc(table_t, W, b):
    bkv = 8192
    return pl.pallas_call(
        _project_body,
        grid=(VOCAB // bkv,),
        in_specs=[
            pl.BlockSpec((DIM, bkv), lambda i: (0, i)),
            pl.BlockSpec((DIM, DIM), lambda i: (0, 0)),
            pl.BlockSpec((1, DIM), lambda i: (0, 0)),
        ],
        out_specs=pl.BlockSpec((bkv, 2 * DIM), lambda i: (i, 0)),
        out_shape=jax.ShapeDtypeStruct((VOCAB, 2 * DIM), jnp.float32),
    )(table_t, W, b.reshape(1, DIM))


def _gather_sc(idx_flat, p_pad):
    mesh = plsc.VectorSubcoreMesh(core_axis_name="c", subcore_axis_name="s")

    @functools.partial(
        pl.kernel,
        mesh=mesh,
        compiler_params=pltpu.CompilerParams(use_tc_tiling_on_sc=True),
        out_type=jax.ShapeDtypeStruct((NROWS, 2 * DIM), jnp.float32),
        scratch_types=[
            pltpu.VMEM((ROWS_PER_W,), jnp.int32),
            pltpu.VMEM((CHUNK, 2 * DIM), jnp.float32),
            pltpu.SemaphoreType.DMA,
        ],
    )
    def k(idx_hbm, tab_hbm, out_hbm, idx_v, rows_v, sem):
        wid = lax.axis_index("s") * NC + lax.axis_index("c")
        base = wid * ROWS_PER_W
        pltpu.sync_copy(idx_hbm.at[pl.ds(base, ROWS_PER_W)], idx_v)
        for j in range(NCHUNK):
            n0 = base + j * CHUNK
            pltpu.async_copy(
                tab_hbm.at[idx_v.at[pl.ds(j * CHUNK, CHUNK)]], rows_v, sem
            ).wait()
            pltpu.sync_copy(rows_v, out_hbm.at[pl.ds(n0, CHUNK)])

    return k(idx_flat, p_pad)


def _transpose_body(x_ref, oa_ref):
    oa_ref[0, :, :] = x_ref[:, 0:DIM].T


def _transpose_tc(feats3):
    bkb = 4096
    nj = BATCH // bkb
    return pl.pallas_call(
        _transpose_body,
        grid=(FIELDS, nj),
        in_specs=[
            pl.BlockSpec((bkb, 2 * DIM), lambda k, j: (k * nj + j, 0)),
        ],
        out_specs=pl.BlockSpec((1, DIM, bkb), lambda k, j: (k, 0, j)),
        out_shape=jax.ShapeDtypeStruct((FIELDS, DIM, BATCH), jnp.float32),
    )(feats3)


def kernel(indices, table, W, b):
    idx_f_major = jnp.transpose(indices).reshape(-1).astype(jnp.int32)
    p_pad = _project_tc(jnp.transpose(table), W, b)
    feats3 = _gather_sc(idx_f_major, p_pad)
    out_feb = _transpose_tc(feats3)
    return jnp.transpose(out_feb, (2, 0, 1))


# speedup vs baseline: 2.4017x; 1.0176x over previous
import functools

import jax
import jax.numpy as jnp
from jax import lax
from jax.experimental import pallas as pl
from jax.experimental.pallas import tpu as pltpu
from jax.experimental.pallas import tpu_sc as plsc

BATCH = 16384
FIELDS = 26
DIM = 64
VOCAB = 1000000
NROWS = BATCH * FIELDS
NPAIR = NROWS // 2

_INFO = plsc.get_sparse_core_info()
NC = _INFO.num_cores
NS = _INFO.num_subcores
NW = NC * NS
ROWS_PER_W = NROWS // NW
CHUNK = 256
NCHUNK = ROWS_PER_W // CHUNK


def _project_body(xt_ref, w_ref, b_ref, o_ref):
    y = jax.lax.dot_general(
        xt_ref[...], w_ref[...], (((0,), (0,)), ((), ())),
        preferred_element_type=jnp.float32,
    )
    o_ref[:, 0:DIM] = y + b_ref[...]
    o_ref[:, DIM:2 * DIM] = jnp.zeros_like(y)


def _project_tc(table_t, W, b):
    bkv = 8192
    return pl.pallas_call(
        _project_body,
        grid=(VOCAB // bkv,),
        in_specs=[
            pl.BlockSpec((DIM, bkv), lambda i: (0, i)),
            pl.BlockSpec((DIM, DIM), lambda i: (0, 0)),
            pl.BlockSpec((1, DIM), lambda i: (0, 0)),
        ],
        out_specs=pl.BlockSpec((bkv, 2 * DIM), lambda i: (i, 0)),
        out_shape=jax.ShapeDtypeStruct((VOCAB, 2 * DIM), jnp.float32),
    )(table_t, W, b.reshape(1, DIM))


def _gather_sc(idx_flat, p_pad):
    mesh = plsc.VectorSubcoreMesh(core_axis_name="c", subcore_axis_name="s")

    @functools.partial(
        pl.kernel,
        mesh=mesh,
        compiler_params=pltpu.CompilerParams(use_tc_tiling_on_sc=True),
        out_type=jax.ShapeDtypeStruct((NROWS, 2 * DIM), jnp.float32),
        scratch_types=[
            pltpu.VMEM((ROWS_PER_W,), jnp.int32),
            pltpu.VMEM((CHUNK, 2 * DIM), jnp.float32),
            pltpu.VMEM((CHUNK, 2 * DIM), jnp.float32),
            pltpu.SemaphoreType.DMA,
            pltpu.SemaphoreType.DMA,
            pltpu.SemaphoreType.DMA,
            pltpu.SemaphoreType.DMA,
        ],
    )
    def k(idx_hbm, tab_hbm, out_hbm, idx_v, rows_a, rows_b, ga, gb, wa, wb):
        wid = lax.axis_index("s") * NC + lax.axis_index("c")
        base = wid * ROWS_PER_W
        pltpu.sync_copy(idx_hbm.at[pl.ds(base, ROWS_PER_W)], idx_v)
        bufs = (rows_a, rows_b)
        gsems = (ga, gb)
        wsems = (wa, wb)
        gd = [None, None]
        wd = [None, None]
        for j in range(NCHUNK):
            bsel = j % 2
            if wd[bsel] is not None:
                wd[bsel].wait()
            gd[bsel] = pltpu.async_copy(
                tab_hbm.at[idx_v.at[pl.ds(j * CHUNK, CHUNK)]],
                bufs[bsel], gsems[bsel],
            )
            if j >= 1:
                p = (j - 1) % 2
                gd[p].wait()
                wd[p] = pltpu.async_copy(
                    bufs[p], out_hbm.at[pl.ds(base + (j - 1) * CHUNK, CHUNK)],
                    wsems[p],
                )
        last = NCHUNK - 1
        bsel = last % 2
        gd[bsel].wait()
        pltpu.sync_copy(bufs[bsel], out_hbm.at[pl.ds(base + last * CHUNK, CHUNK)])
        wd[(last - 1) % 2].wait()

    return k(idx_flat, p_pad)


def _transpose_body(x_ref, oa_ref):
    oa_ref[0, :, :] = x_ref[:, 0:DIM].T


def _transpose_tc(feats3):
    bkb = 4096
    nj = BATCH // bkb
    return pl.pallas_call(
        _transpose_body,
        grid=(FIELDS, nj),
        in_specs=[
            pl.BlockSpec((bkb, 2 * DIM), lambda k, j: (k * nj + j, 0)),
        ],
        out_specs=pl.BlockSpec((1, DIM, bkb), lambda k, j: (k, 0, j)),
        out_shape=jax.ShapeDtypeStruct((FIELDS, DIM, BATCH), jnp.float32),
    )(feats3)


def kernel(indices, table, W, b):
    idx_f_major = jnp.transpose(indices).reshape(-1).astype(jnp.int32)
    p_pad = _project_tc(jnp.transpose(table), W, b)
    feats3 = _gather_sc(idx_f_major, p_pad)
    out_feb = _transpose_tc(feats3)
    return jnp.transpose(out_feb, (2, 0, 1))


# speedup vs baseline: 2.4962x; 1.0394x over previous
import functools

import jax
import jax.numpy as jnp
from jax import lax
from jax.experimental import pallas as pl
from jax.experimental.pallas import tpu as pltpu
from jax.experimental.pallas import tpu_sc as plsc

BATCH = 16384
FIELDS = 26
DIM = 64
VOCAB = 1000000
NROWS = BATCH * FIELDS
NPAIR = NROWS // 2

_INFO = plsc.get_sparse_core_info()
NC = _INFO.num_cores
NS = _INFO.num_subcores
NW = NC * NS
NSPLIT = 2
HROWS = NROWS // NSPLIT
HFIELDS = FIELDS // NSPLIT
ROWS_PER_W = HROWS // NW
CHUNK = 256
NCHUNK = ROWS_PER_W // CHUNK


def _project_body(xt_ref, w_ref, b_ref, o_ref):
    y = jax.lax.dot_general(
        xt_ref[...], w_ref[...], (((0,), (0,)), ((), ())),
        preferred_element_type=jnp.float32,
    )
    o_ref[:, 0:DIM] = y + b_ref[...]
    o_ref[:, DIM:2 * DIM] = jnp.zeros_like(y)


def _project_tc(table_t, W, b):
    bkv = 8192
    return pl.pallas_call(
        _project_body,
        grid=(VOCAB // bkv,),
        in_specs=[
            pl.BlockSpec((DIM, bkv), lambda i: (0, i)),
            pl.BlockSpec((DIM, DIM), lambda i: (0, 0)),
            pl.BlockSpec((1, DIM), lambda i: (0, 0)),
        ],
        out_specs=pl.BlockSpec((bkv, 2 * DIM), lambda i: (i, 0)),
        out_shape=jax.ShapeDtypeStruct((VOCAB, 2 * DIM), jnp.float32),
    )(table_t, W, b.reshape(1, DIM))


def _gather_sc(idx_flat, p_pad):
    mesh = plsc.VectorSubcoreMesh(core_axis_name="c", subcore_axis_name="s")

    @functools.partial(
        pl.kernel,
        mesh=mesh,
        compiler_params=pltpu.CompilerParams(use_tc_tiling_on_sc=True),
        out_type=jax.ShapeDtypeStruct((HROWS, 2 * DIM), jnp.float32),
        scratch_types=[
            pltpu.VMEM((ROWS_PER_W,), jnp.int32),
            pltpu.VMEM((CHUNK, 2 * DIM), jnp.float32),
            pltpu.VMEM((CHUNK, 2 * DIM), jnp.float32),
            pltpu.SemaphoreType.DMA,
            pltpu.SemaphoreType.DMA,
            pltpu.SemaphoreType.DMA,
            pltpu.SemaphoreType.DMA,
        ],
    )
    def k(idx_hbm, tab_hbm, out_hbm, idx_v, rows_a, rows_b, ga, gb, wa, wb):
        wid = lax.axis_index("s") * NC + lax.axis_index("c")
        base = wid * ROWS_PER_W
        pltpu.sync_copy(idx_hbm.at[pl.ds(base, ROWS_PER_W)], idx_v)
        bufs = (rows_a, rows_b)
        gsems = (ga, gb)
        wsems = (wa, wb)
        gd = [None, None]
        wd = [None, None]
        for j in range(NCHUNK):
            bsel = j % 2
            if wd[bsel] is not None:
                wd[bsel].wait()
            gd[bsel] = pltpu.async_copy(
                tab_hbm.at[idx_v.at[pl.ds(j * CHUNK, CHUNK)]],
                bufs[bsel], gsems[bsel],
            )
            if j >= 1:
                p = (j - 1) % 2
                gd[p].wait()
                wd[p] = pltpu.async_copy(
                    bufs[p], out_hbm.at[pl.ds(base + (j - 1) * CHUNK, CHUNK)],
                    wsems[p],
                )
        last = NCHUNK - 1
        bsel = last % 2
        gd[bsel].wait()
        pltpu.sync_copy(bufs[bsel], out_hbm.at[pl.ds(base + last * CHUNK, CHUNK)])
        wd[(last - 1) % 2].wait()

    return k(idx_flat, p_pad)


def _transpose_body(x_ref, oa_ref):
    oa_ref[0, :, :] = x_ref[:, 0:DIM].T


def _transpose_body_alias(x_ref, d_ref, oa_ref):
    del d_ref
    oa_ref[0, :, :] = x_ref[:, 0:DIM].T


def _transpose_tc(feats_h, donor, foff):
    bkb = 4096
    nj = BATCH // bkb
    out_shape = jax.ShapeDtypeStruct((FIELDS, DIM, BATCH), jnp.float32)
    x_spec = pl.BlockSpec((bkb, 2 * DIM), lambda k, j: (k * nj + j, 0))
    o_spec = pl.BlockSpec((1, DIM, bkb), lambda k, j: (k + foff, 0, j))
    if donor is None:
        return pl.pallas_call(
            _transpose_body,
            grid=(HFIELDS, nj),
            in_specs=[x_spec],
            out_specs=o_spec,
            out_shape=out_shape,
        )(feats_h)
    return pl.pallas_call(
        _transpose_body_alias,
        grid=(HFIELDS, nj),
        in_specs=[x_spec, pl.BlockSpec(memory_space=pl.ANY)],
        out_specs=o_spec,
        out_shape=out_shape,
        input_output_aliases={1: 0},
    )(feats_h, donor)


def kernel(indices, table, W, b):
    idx_f_major = jnp.transpose(indices).reshape(-1).astype(jnp.int32)
    p_pad = _project_tc(jnp.transpose(table), W, b)
    y = None
    for h in range(NSPLIT):
        idx_h = lax.slice(idx_f_major, (h * HROWS,), ((h + 1) * HROWS,))
        feats_h = _gather_sc(idx_h, p_pad)
        y = _transpose_tc(feats_h, y, h * HFIELDS)
    return jnp.transpose(y, (2, 0, 1))
